# baseline (speedup 1.0000x reference)
import functools

import jax
import jax.numpy as jnp
from jax import lax
from jax.experimental import pallas as pl
from jax.experimental.pallas import tpu as pltpu
from jax.experimental.pallas import tpu_sc as plsc

INP_DIM = 64
TAG_DIM = 16
OUT_DIM = INP_DIM + 2 * TAG_DIM
PAD_DIM = 128
CHUNK = 128
K = 5
SB = 8
BB = 4


@functools.cache
def _build(s_len: int, b_len: int):
    info = plsc.get_sparse_core_info()
    nw = info.num_cores * info.num_subcores
    assert nw == SB * BB
    s_blk = s_len // SB
    b_blk = b_len // BB
    per_w = s_blk * b_blk
    n_chunks = per_w // CHUNK
    n_blocks = n_chunks // K
    halves = b_blk // CHUNK
    n_rows = s_len * b_len

    mesh = plsc.VectorSubcoreMesh(core_axis_name="c", subcore_axis_name="s")

    scratch = (
        [pltpu.VMEM((s_blk, b_blk), jnp.int32)] * 3
        + [pltpu.VMEM((CHUNK, INP_DIM), jnp.float32) for _ in range(K)]
        + [pltpu.VMEM((CHUNK, TAG_DIM), jnp.float32) for _ in range(2 * K)]
        + [pltpu.SemaphoreType.DMA for _ in range(2 * K)]
    )

    @functools.partial(
        pl.kernel,
        mesh=mesh,
        out_type=jax.ShapeDtypeStruct((n_rows, PAD_DIM), jnp.float32),
        scratch_types=scratch,
        compiler_params=pltpu.CompilerParams(use_tc_tiling_on_sc=False),
    )
    def k(widx_hbm, pidx_hbm, nidx_hbm, wtab_hbm, ptab_hbm, ntab_hbm,
          out_hbm, widx_v, pidx_v, nidx_v, *bufs):
        wrows = bufs[:K]
        prows = bufs[K:2 * K]
        nrows = bufs[2 * K:3 * K]
        gsem = bufs[3 * K:4 * K]
        osem = bufs[4 * K:5 * K]

        wid = lax.axis_index("s") * info.num_cores + lax.axis_index("c")
        si = (wid // BB) * s_blk
        bj = (wid % BB) * b_blk
        blk = (pl.ds(si, s_blk), pl.ds(bj, b_blk))
        pltpu.sync_copy(widx_hbm.at[blk[0], blk[1]], widx_v)
        pltpu.sync_copy(pidx_hbm.at[blk[0], blk[1]], pidx_v)
        pltpu.sync_copy(nidx_hbm.at[blk[0], blk[1]], nidx_v)

        def body(g, carry):
            gathers = []
            for b in range(K):
                c = g * K + b
                r = c // halves
                off = (c % halves) * CHUNK
                gathers.append((
                    pltpu.async_copy(
                        wtab_hbm.at[widx_v.at[r, pl.ds(off, CHUNK)]],
                        wrows[b], gsem[b]),
                    pltpu.async_copy(
                        ptab_hbm.at[pidx_v.at[r, pl.ds(off, CHUNK)]],
                        prows[b], gsem[b]),
                    pltpu.async_copy(
                        ntab_hbm.at[nidx_v.at[r, pl.ds(off, CHUNK)]],
                        nrows[b], gsem[b]),
                ))
            writes = []
            for b in range(K):
                for d in gathers[b]:
                    d.wait()
                c = g * K + b
                r = c // halves
                off = (c % halves) * CHUNK
                base = (si + r) * b_len + bj + off
                writes.append((
                    pltpu.async_copy(
                        wrows[b],
                        out_hbm.at[pl.ds(base, CHUNK), pl.ds(0, INP_DIM)],
                        osem[b]),
                    pltpu.async_copy(
                        prows[b],
                        out_hbm.at[pl.ds(base, CHUNK),
                                   pl.ds(INP_DIM, TAG_DIM)],
                        osem[b]),
                    pltpu.async_copy(
                        nrows[b],
                        out_hbm.at[pl.ds(base, CHUNK),
                                   pl.ds(INP_DIM + TAG_DIM, TAG_DIM)],
                        osem[b]),
                ))
            for b in range(K):
                for d in writes[b]:
                    d.wait()
            return carry

        lax.fori_loop(0, n_blocks, body, 0, unroll=False)

    return k


def _tc_slice(x, s_len, b_len):
    def body(i_ref, o_ref):
        o_ref[...] = i_ref[:, :OUT_DIM].reshape(1, b_len, OUT_DIM)

    return pl.pallas_call(
        body,
        grid=(s_len,),
        in_specs=[pl.BlockSpec((b_len, PAD_DIM), lambda i: (i, 0))],
        out_specs=pl.BlockSpec((1, b_len, OUT_DIM), lambda i: (i, 0, 0)),
        out_shape=jax.ShapeDtypeStruct((s_len, b_len, OUT_DIM), jnp.float32),
    )(x)


def kernel(seq_word, seq_pos, seq_ner, word_table, pos_table, ner_table):
    s, b = seq_word.shape
    run = _build(s, b)
    out = run(
        seq_word.astype(jnp.int32), seq_pos.astype(jnp.int32),
        seq_ner.astype(jnp.int32), word_table, pos_table, ner_table)
    return _tc_slice(out, s, b)


# speedup vs baseline: 1.1563x; 1.1563x over previous
import functools

import jax
import jax.numpy as jnp
from jax import lax
from jax.experimental import pallas as pl
from jax.experimental.pallas import tpu as pltpu
from jax.experimental.pallas import tpu_sc as plsc

INP_DIM = 64
TAG_DIM = 16
OUT_DIM = INP_DIM + 2 * TAG_DIM
PAD_DIM = 128
CHUNK = 128
K = 5
SB = 8
BB = 4


@functools.cache
def _build(s_len: int, b_len: int):
    info = plsc.get_sparse_core_info()
    nw = info.num_cores * info.num_subcores
    assert nw == SB * BB
    s_blk = s_len // SB
    b_blk = b_len // BB
    per_w = s_blk * b_blk
    n_chunks = per_w // CHUNK
    n_blocks = n_chunks // K
    halves = b_blk // CHUNK
    n_rows = s_len * b_len

    mesh = plsc.VectorSubcoreMesh(core_axis_name="c", subcore_axis_name="s")

    scratch = (
        [pltpu.VMEM((s_blk, b_blk), jnp.int32)] * 3
        + [pltpu.VMEM((CHUNK, INP_DIM), jnp.float32) for _ in range(K)]
        + [pltpu.VMEM((CHUNK, TAG_DIM), jnp.float32) for _ in range(2 * K)]
        + [pltpu.SemaphoreType.DMA for _ in range(2 * K)]
    )

    @functools.partial(
        pl.kernel,
        mesh=mesh,
        out_type=jax.ShapeDtypeStruct((n_rows, PAD_DIM), jnp.float32),
        scratch_types=scratch,
        compiler_params=pltpu.CompilerParams(use_tc_tiling_on_sc=False),
    )
    def k(widx_hbm, pidx_hbm, nidx_hbm, wtab_hbm, ptab_hbm, ntab_hbm,
          out_hbm, widx_v, pidx_v, nidx_v, *bufs):
        wrows = bufs[:K]
        prows = bufs[K:2 * K]
        nrows = bufs[2 * K:3 * K]
        gsem = bufs[3 * K:4 * K]
        osem = bufs[4 * K:5 * K]

        wid = lax.axis_index("s") * info.num_cores + lax.axis_index("c")
        si = (wid // BB) * s_blk
        bj = (wid % BB) * b_blk
        pltpu.sync_copy(widx_hbm.at[pl.ds(si, s_blk), pl.ds(bj, b_blk)],
                        widx_v)
        pltpu.sync_copy(pidx_hbm.at[pl.ds(si, s_blk), pl.ds(bj, b_blk)],
                        pidx_v)
        pltpu.sync_copy(nidx_hbm.at[pl.ds(si, s_blk), pl.ds(bj, b_blk)],
                        nidx_v)

        def body(g, carry):
            gathers = []
            for b in range(K):
                c = g * K + b
                r = c // halves
                off = (c % halves) * CHUNK
                gathers.append((
                    pltpu.async_copy(
                        wtab_hbm.at[widx_v.at[r, pl.ds(off, CHUNK)]],
                        wrows[b], gsem[b]),
                    pltpu.async_copy(
                        ptab_hbm.at[pidx_v.at[r, pl.ds(off, CHUNK)]],
                        prows[b], gsem[b]),
                    pltpu.async_copy(
                        ntab_hbm.at[nidx_v.at[r, pl.ds(off, CHUNK)]],
                        nrows[b], gsem[b]),
                ))
            writes = []
            for b in range(K):
                for d in gathers[b]:
                    d.wait()
                c = g * K + b
                r = c // halves
                off = (c % halves) * CHUNK
                base = (si + r) * b_len + bj + off
                writes.append((
                    pltpu.async_copy(
                        wrows[b],
                        out_hbm.at[pl.ds(base, CHUNK), pl.ds(0, INP_DIM)],
                        osem[b]),
                    pltpu.async_copy(
                        prows[b],
                        out_hbm.at[pl.ds(base, CHUNK),
                                   pl.ds(INP_DIM, TAG_DIM)],
                        osem[b]),
                    pltpu.async_copy(
                        nrows[b],
                        out_hbm.at[pl.ds(base, CHUNK),
                                   pl.ds(INP_DIM + TAG_DIM, TAG_DIM)],
                        osem[b]),
                ))
            for b in range(K):
                for d in writes[b]:
                    d.wait()
            return carry

        lax.fori_loop(0, n_blocks, body, 0, unroll=False)

    return k


_REPACK_ROWS = 8000


def _tc_repack(tab):
    v = tab.shape[0]
    r = _REPACK_ROWS
    h = v // 2

    def body(t_ref, b_ref, o_ref):
        o_ref[...] = jnp.concatenate([t_ref[...], b_ref[...]], axis=1)

    out = pl.pallas_call(
        body,
        grid=(h // r,),
        in_specs=[
            pl.BlockSpec((r, INP_DIM), lambda i: (i, 0)),
            pl.BlockSpec((r, INP_DIM), lambda i: (i + h // r, 0)),
        ],
        out_specs=pl.BlockSpec((r, 2 * INP_DIM), lambda i: (i, 0)),
        out_shape=jax.ShapeDtypeStruct((h, 2 * INP_DIM), jnp.float32),
    )(tab, tab)
    return out.reshape(v, INP_DIM)


def kernel(seq_word, seq_pos, seq_ner, word_table, pos_table, ner_table):
    s, b = seq_word.shape
    v = word_table.shape[0]
    h = v // 2
    wv = seq_word.astype(jnp.int32)
    widx = 2 * wv - jnp.where(wv >= h, 2 * h - 1, 0).astype(jnp.int32)
    run = _build(s, b)
    out = run(
        widx, seq_pos.astype(jnp.int32), seq_ner.astype(jnp.int32),
        _tc_repack(word_table), pos_table, ner_table)
    return out.reshape(s, b, PAD_DIM)[:, :, :OUT_DIM]
